# baseline (speedup 1.0000x reference)
import functools

import jax
import jax.numpy as jnp
from jax import lax
from jax.experimental import pallas as pl
from jax.experimental.pallas import tpu as pltpu
from jax.experimental.pallas import tpu_sc as plsc

N = 10000
NP = 10240
E = 320000
D = 128
ALPHA = 0.1
EPS = 1e-6

NC = 2
NS = 16
EP = 327680
NPAD = EP - E
ROWS_PER_TILE = NP // NS

W = 128
NROW = EP // W
ROWS_PER_SC = NROW // NC
ROWS_PER_TILE_E = ROWS_PER_SC // NS
CH = 16
NCHUNK = ROWS_PER_TILE_E // CH

WG = 64
NROWG = EP // WG
ROWS_PER_SC_G = NROWG // NC
ROWS_PER_TILE_G = ROWS_PER_SC_G // NS
CHG = 16
NCHUNK_G = ROWS_PER_TILE_G // CHG
NBUF = 3


def _agg_body(src_hbm, dst_hbm, hp_hbm, zeros_hbm, out_hbm, acc, sidx, didx,
              rows0, rows1, rows2, sem0, sem1, sem2):
    c = lax.axis_index("c")
    s = lax.axis_index("s")
    r0 = s * ROWS_PER_TILE
    pltpu.sync_copy(zeros_hbm.at[pl.ds(r0, ROWS_PER_TILE)],
                    acc.at[pl.ds(r0, ROWS_PER_TILE)])
    plsc.subcore_barrier()
    rows = (rows0, rows1, rows2)
    sems = (sem0, sem1, sem2)

    def chunk(k, carry):
        row0 = c * ROWS_PER_SC_G + s * ROWS_PER_TILE_G + k * CHG
        pltpu.sync_copy(src_hbm.at[pl.ds(row0, CHG)], sidx)
        pltpu.sync_copy(dst_hbm.at[pl.ds(row0, CHG)], didx)
        pend = [None] * CHG
        for j in range(min(2, CHG)):
            pend[j] = pltpu.async_copy(hp_hbm.at[sidx.at[j]],
                                       rows[j % NBUF], sems[j % NBUF])
        for j in range(CHG):
            if j + 2 < CHG:
                pend[j + 2] = pltpu.async_copy(hp_hbm.at[sidx.at[j + 2]],
                                               rows[(j + 2) % NBUF],
                                               sems[(j + 2) % NBUF])
            pend[j].wait()
            pltpu.sync_copy(rows[j % NBUF], acc.at[didx.at[j]], add=True)
        return carry

    lax.fori_loop(0, NCHUNK_G, chunk, None)
    plsc.subcore_barrier()
    pltpu.sync_copy(acc.at[pl.ds(r0, ROWS_PER_TILE)],
                    out_hbm.at[c, pl.ds(r0, ROWS_PER_TILE)])


def _cnt_body(dst_hbm, ones_hbm, zeros_hbm, out_hbm, acc, didx, ones, sem):
    c = lax.axis_index("c")
    s = lax.axis_index("s")
    r0 = s * ROWS_PER_TILE
    pltpu.sync_copy(zeros_hbm.at[pl.ds(r0, ROWS_PER_TILE)],
                    acc.at[pl.ds(r0, ROWS_PER_TILE)])
    pltpu.sync_copy(ones_hbm, ones)
    plsc.subcore_barrier()

    def chunk(k, carry):
        row0 = c * ROWS_PER_SC + s * ROWS_PER_TILE_E + k * CH
        pltpu.sync_copy(dst_hbm.at[pl.ds(row0, CH)], didx)
        ds = [pltpu.async_copy(ones, acc.at[didx.at[j]], sem, add=True)
              for j in range(CH)]
        for d in ds:
            d.wait()
        return carry

    lax.fori_loop(0, NCHUNK, chunk, None)
    plsc.subcore_barrier()
    pltpu.sync_copy(acc.at[pl.ds(r0, ROWS_PER_TILE)],
                    out_hbm.at[c, pl.ds(r0, ROWS_PER_TILE)])


@functools.lru_cache(maxsize=None)
def _sc_kernels():
    mesh = plsc.VectorSubcoreMesh(core_axis_name="c", subcore_axis_name="s",
                                  num_cores=NC, num_subcores=NS)
    agg_kernel = pl.kernel(
        _agg_body,
        out_type=jax.ShapeDtypeStruct((NC, NP, D), jnp.float32),
        mesh=mesh,
        scratch_types=[
            pltpu.VMEM_SHARED((NP, D), jnp.float32),
            pltpu.VMEM((CHG, WG), jnp.int32),
            pltpu.VMEM((CHG, WG), jnp.int32),
            pltpu.VMEM((WG, D), jnp.float32),
            pltpu.VMEM((WG, D), jnp.float32),
            pltpu.VMEM((WG, D), jnp.float32),
            pltpu.SemaphoreType.DMA,
            pltpu.SemaphoreType.DMA,
            pltpu.SemaphoreType.DMA,
        ],
    )
    cnt_kernel = pl.kernel(
        _cnt_body,
        out_type=jax.ShapeDtypeStruct((NC, NP, D), jnp.float32),
        mesh=mesh,
        scratch_types=[
            pltpu.VMEM_SHARED((NP, D), jnp.float32),
            pltpu.VMEM((CH, W), jnp.int32),
            pltpu.VMEM((W, D), jnp.float32),
            pltpu.SemaphoreType.DMA,
        ],
    )
    return cnt_kernel, agg_kernel


def _dis_from_parts(degp):
    deg = degp[0, :N, 0:1] + degp[1, :N, 0:1] + 1.0
    return lax.rsqrt(deg)


def _hprime_body(x_ref, wt_ref, b_ref, degp_ref, out_ref):
    dis = _dis_from_parts(degp_ref[...])
    h = jnp.dot(x_ref[...], wt_ref[...], preferred_element_type=jnp.float32)
    out_ref[:N, :] = dis * (h + b_ref[...])
    out_ref[N:, :] = jnp.zeros((NP - N, D), jnp.float32)


def _fin_body(agg_ref, hp_ref, degp_ref, x0_ref, w1t_ref, w2t_ref, g_ref,
              b_ref, out_ref):
    dis = _dis_from_parts(degp_ref[...])
    ax = (agg_ref[0, :N, :] + agg_ref[1, :N, :] + hp_ref[:N, :]) * dis
    out = ((1.0 - ALPHA) * jnp.dot(ax, w1t_ref[...],
                                   preferred_element_type=jnp.float32)
           + ALPHA * jnp.dot(x0_ref[...], w2t_ref[...],
                             preferred_element_type=jnp.float32))
    msq = jnp.mean(out * out, axis=0, keepdims=True)
    rms = jnp.sqrt(msq + EPS)
    out_ref[...] = jnp.maximum(g_ref[...] * (out / rms) + b_ref[...], 0.0)


def kernel(x, x0, edge_index, W_gcn, b_gcn, W1, W2, gamma, beta):
    pad = N + (jnp.arange(NPAD, dtype=jnp.int32) % (NP - N))
    src = jnp.concatenate([edge_index[0].astype(jnp.int32), pad])
    dst = jnp.concatenate([edge_index[1].astype(jnp.int32), pad])
    srcg = src.reshape(NROWG, WG)
    dstg = dst.reshape(NROWG, WG)
    dst = dst.reshape(NROW, W)
    zeros_nd = jnp.zeros((NP, D), jnp.float32)
    ones_w = jnp.ones((W, D), jnp.float32)

    cnt_kernel, agg_kernel = _sc_kernels()
    degp = cnt_kernel(dst, ones_w, zeros_nd)

    hp = pl.pallas_call(
        _hprime_body,
        out_shape=jax.ShapeDtypeStruct((NP, D), jnp.float32),
    )(x, W_gcn.T, b_gcn.reshape(1, D), degp)

    agg = agg_kernel(srcg, dstg, hp, zeros_nd)

    out = pl.pallas_call(
        _fin_body,
        out_shape=jax.ShapeDtypeStruct((N, D), jnp.float32),
    )(agg, hp, degp, x0, W1.T, W2.T, gamma.reshape(1, D), beta.reshape(1, D))

    return (out, x0, edge_index)


# speedup vs baseline: 1.0272x; 1.0272x over previous
import functools

import jax
import jax.numpy as jnp
from jax import lax
from jax.experimental import pallas as pl
from jax.experimental.pallas import tpu as pltpu
from jax.experimental.pallas import tpu_sc as plsc

N = 10000
NP = 10240
E = 320000
D = 128
ALPHA = 0.1
EPS = 1e-6

NC = 2
NS = 16
EP = 327680
NPAD = EP - E
ROWS_PER_TILE = NP // NS

W = 128
NROW = EP // W
ROWS_PER_SC = NROW // NC
ROWS_PER_TILE_E = ROWS_PER_SC // NS
CH = 16
NCHUNK = ROWS_PER_TILE_E // CH

WG = 128
NROWG = EP // WG
ROWS_PER_SC_G = NROWG // NC
ROWS_PER_TILE_G = ROWS_PER_SC_G // NS
CHG = 16
NCHUNK_G = ROWS_PER_TILE_G // CHG
NBUF = 2
DEPTH = NBUF - 1


def _agg_body(src_hbm, dst_hbm, hp_hbm, zeros_hbm, out_hbm, acc, sidx, didx,
              rows0, rows1, sem0, sem1):
    c = lax.axis_index("c")
    s = lax.axis_index("s")
    r0 = s * ROWS_PER_TILE
    pltpu.sync_copy(zeros_hbm.at[pl.ds(r0, ROWS_PER_TILE)],
                    acc.at[pl.ds(r0, ROWS_PER_TILE)])
    plsc.subcore_barrier()
    rows = (rows0, rows1)
    sems = (sem0, sem1)

    def chunk(k, carry):
        row0 = c * ROWS_PER_SC_G + s * ROWS_PER_TILE_G + k * CHG
        pltpu.sync_copy(src_hbm.at[pl.ds(row0, CHG)], sidx)
        pltpu.sync_copy(dst_hbm.at[pl.ds(row0, CHG)], didx)
        pend = [None] * CHG
        for j in range(min(DEPTH, CHG)):
            pend[j] = pltpu.async_copy(hp_hbm.at[sidx.at[j]],
                                       rows[j % NBUF], sems[j % NBUF])
        for j in range(CHG):
            if j + DEPTH < CHG:
                pend[j + DEPTH] = pltpu.async_copy(
                    hp_hbm.at[sidx.at[j + DEPTH]],
                    rows[(j + DEPTH) % NBUF], sems[(j + DEPTH) % NBUF])
            pend[j].wait()
            pltpu.sync_copy(rows[j % NBUF], acc.at[didx.at[j]], add=True)
        return carry

    lax.fori_loop(0, NCHUNK_G, chunk, None)
    plsc.subcore_barrier()
    pltpu.sync_copy(acc.at[pl.ds(r0, ROWS_PER_TILE)],
                    out_hbm.at[c, pl.ds(r0, ROWS_PER_TILE)])


def _cnt_body(dst_hbm, ones_hbm, zeros_hbm, out_hbm, acc, didx, ones, sem):
    c = lax.axis_index("c")
    s = lax.axis_index("s")
    r0 = s * ROWS_PER_TILE
    pltpu.sync_copy(zeros_hbm.at[pl.ds(r0, ROWS_PER_TILE)],
                    acc.at[pl.ds(r0, ROWS_PER_TILE)])
    pltpu.sync_copy(ones_hbm, ones)
    plsc.subcore_barrier()

    def chunk(k, carry):
        row0 = c * ROWS_PER_SC + s * ROWS_PER_TILE_E + k * CH
        pltpu.sync_copy(dst_hbm.at[pl.ds(row0, CH)], didx)
        ds = [pltpu.async_copy(ones, acc.at[didx.at[j]], sem, add=True)
              for j in range(CH)]
        for d in ds:
            d.wait()
        return carry

    lax.fori_loop(0, NCHUNK, chunk, None)
    plsc.subcore_barrier()
    pltpu.sync_copy(acc.at[pl.ds(r0, ROWS_PER_TILE)],
                    out_hbm.at[c, pl.ds(r0, ROWS_PER_TILE)])


@functools.lru_cache(maxsize=None)
def _sc_kernels():
    mesh = plsc.VectorSubcoreMesh(core_axis_name="c", subcore_axis_name="s",
                                  num_cores=NC, num_subcores=NS)
    agg_kernel = pl.kernel(
        _agg_body,
        out_type=jax.ShapeDtypeStruct((NC, NP, D), jnp.float32),
        mesh=mesh,
        scratch_types=[
            pltpu.VMEM_SHARED((NP, D), jnp.float32),
            pltpu.VMEM((CHG, WG), jnp.int32),
            pltpu.VMEM((CHG, WG), jnp.int32),
            pltpu.VMEM((WG, D), jnp.float32),
            pltpu.VMEM((WG, D), jnp.float32),
            pltpu.SemaphoreType.DMA,
            pltpu.SemaphoreType.DMA,
        ],
    )
    cnt_kernel = pl.kernel(
        _cnt_body,
        out_type=jax.ShapeDtypeStruct((NC, NP, D), jnp.float32),
        mesh=mesh,
        scratch_types=[
            pltpu.VMEM_SHARED((NP, D), jnp.float32),
            pltpu.VMEM((CH, W), jnp.int32),
            pltpu.VMEM((W, D), jnp.float32),
            pltpu.SemaphoreType.DMA,
        ],
    )
    return cnt_kernel, agg_kernel


def _dis_from_parts(degp):
    deg = degp[0, :N, 0:1] + degp[1, :N, 0:1] + 1.0
    return lax.rsqrt(deg)


def _hprime_body(x_ref, wt_ref, b_ref, degp_ref, out_ref):
    dis = _dis_from_parts(degp_ref[...])
    h = jnp.dot(x_ref[...], wt_ref[...], preferred_element_type=jnp.float32)
    out_ref[:N, :] = dis * (h + b_ref[...])
    out_ref[N:, :] = jnp.zeros((NP - N, D), jnp.float32)


def _fin_body(agg_ref, hp_ref, degp_ref, x0_ref, w1t_ref, w2t_ref, g_ref,
              b_ref, out_ref):
    dis = _dis_from_parts(degp_ref[...])
    ax = (agg_ref[0, :N, :] + agg_ref[1, :N, :] + hp_ref[:N, :]) * dis
    out = ((1.0 - ALPHA) * jnp.dot(ax, w1t_ref[...],
                                   preferred_element_type=jnp.float32)
           + ALPHA * jnp.dot(x0_ref[...], w2t_ref[...],
                             preferred_element_type=jnp.float32))
    msq = jnp.mean(out * out, axis=0, keepdims=True)
    rms = jnp.sqrt(msq + EPS)
    out_ref[...] = jnp.maximum(g_ref[...] * (out / rms) + b_ref[...], 0.0)


def kernel(x, x0, edge_index, W_gcn, b_gcn, W1, W2, gamma, beta):
    pad = N + (jnp.arange(NPAD, dtype=jnp.int32) % (NP - N))
    src = jnp.concatenate([edge_index[0].astype(jnp.int32), pad])
    dst = jnp.concatenate([edge_index[1].astype(jnp.int32), pad])
    srcg = src.reshape(NROWG, WG)
    dstg = dst.reshape(NROWG, WG)
    dst = dst.reshape(NROW, W)
    zeros_nd = jnp.zeros((NP, D), jnp.float32)
    ones_w = jnp.ones((W, D), jnp.float32)

    cnt_kernel, agg_kernel = _sc_kernels()
    degp = cnt_kernel(dst, ones_w, zeros_nd)

    hp = pl.pallas_call(
        _hprime_body,
        out_shape=jax.ShapeDtypeStruct((NP, D), jnp.float32),
    )(x, W_gcn.T, b_gcn.reshape(1, D), degp)

    agg = agg_kernel(srcg, dstg, hp, zeros_nd)

    out = pl.pallas_call(
        _fin_body,
        out_shape=jax.ShapeDtypeStruct((N, D), jnp.float32),
    )(agg, hp, degp, x0, W1.T, W2.T, gamma.reshape(1, D), beta.reshape(1, D))

    return (out, x0, edge_index)
